# baseline (speedup 1.0000x reference)
import functools
import jax
import jax.numpy as jnp
from jax import lax
from jax.experimental import pallas as pl
from jax.experimental.pallas import tpu as pltpu
from jax.experimental.pallas import tpu_sc as plsc

_C = 32
_GAMMA = 12.0
_CLIP = 0.01
_BN = 16
_S = 2048
_N = 1024
_CH = 16
_WORKERS = 32


def _lane_lt(width, period, shape):
    lane = lax.broadcasted_iota(jnp.int32, shape, 2)
    return (lane % period) < width


def _reduce_body(x_ref, bw_ref, st_ref):
    sh = (_BN, _C, 128)
    lvl = []
    for j in range(0, _S // 128, 2):
        a = x_ref[:, :, 128 * j:128 * (j + 1)]
        b = x_ref[:, :, 128 * (j + 1):128 * (j + 2)]
        a = jnp.maximum(a, pltpu.roll(a, 128 - 16, 2))
        b = jnp.maximum(b, pltpu.roll(b, 128 - 16, 2))
        lvl.append(jnp.where(_lane_lt(16, 32, sh), a, pltpu.roll(b, 16, 2)))
    w = 16
    while len(lvl) > 1:
        w //= 2
        nxt = []
        for j in range(0, len(lvl), 2):
            a = lvl[j]
            b = lvl[j + 1]
            a = jnp.maximum(a, pltpu.roll(a, 128 - w, 2))
            b = jnp.maximum(b, pltpu.roll(b, 128 - w, 2))
            nxt.append(jnp.where(_lane_lt(w, 2 * w, sh), a,
                                 pltpu.roll(b, w, 2)))
        lvl = nxt
    v = lvl[0]
    while w > 1:
        w //= 2
        v = jnp.maximum(v, pltpu.roll(v, 128 - w, 2))
    for r in (2, 4, 8, 16, 32, 64):
        v = v + pltpu.roll(v, 128 - r, 2)
    s = v[:, :, 0:1].reshape(_BN, _C) * bw_ref[...]
    st_ref[...] = jnp.swapaxes(s, 0, 1)[None]


def _tc_scores_t(routings, boosting_weights):
    n = routings.shape[0]
    x = jnp.transpose(routings, (0, 2, 1))
    bw = boosting_weights.reshape(1, _C)
    return pl.pallas_call(
        _reduce_body,
        grid=(n // _BN,),
        in_specs=[
            pl.BlockSpec((_BN, _C, _S), lambda i: (i, 0, 0)),
            pl.BlockSpec((1, _C), lambda i: (0, 0)),
        ],
        out_specs=pl.BlockSpec((1, _C, _BN), lambda i: (i, 0, 0)),
        out_shape=jax.ShapeDtypeStruct((n // _BN, _C, _BN), jnp.float32),
    )(x, bw)


_TILE = _C * _CH


def _sc_rank_body(st_hbm, mask_hbm, ranks_hbm, s_v, mask_v, ranks_v):
    wid = lax.axis_index("s") * 2 + lax.axis_index("c")
    per_w = (_N // _BN) // _WORKERS

    def tile(i, carry):
        base = (wid * per_w + i) * _TILE
        pltpu.sync_copy(st_hbm.at[pl.ds(base, _TILE)], s_v)
        v = [s_v[pl.ds(c * _CH, _CH)] for c in range(_C)]
        r = [jnp.zeros((_CH,), jnp.float32) for _ in range(_C)]
        u = [jnp.zeros((_CH,), jnp.float32) for _ in range(_C)]
        for c in range(_C):
            for k in range(c):
                t = jnp.maximum(-jnp.sign(v[k] - v[c]), 0.0)
                r[k] = r[k] + t
                u[c] = u[c] + t
        for c in range(_C):
            rc = r[c] + (float(c) - u[c])
            mk = jnp.exp(rc * (-_GAMMA / (_C - 1)))
            mk = mk * (1.0 - jnp.clip(rc - 11.0, 0.0, 1.0))
            mask_v[pl.ds(c * _CH, _CH)] = mk
            ranks_v[pl.ds(c * _CH, _CH)] = rc
        pltpu.sync_copy(mask_v, mask_hbm.at[pl.ds(base, _TILE)])
        pltpu.sync_copy(ranks_v, ranks_hbm.at[pl.ds(base, _TILE)])
        return carry

    lax.fori_loop(0, per_w, tile, 0)


@functools.partial(
    pl.kernel,
    mesh=plsc.VectorSubcoreMesh(core_axis_name="c", subcore_axis_name="s"),
    out_type=[
        jax.ShapeDtypeStruct((_N * _C,), jnp.float32),
        jax.ShapeDtypeStruct((_N * _C,), jnp.float32),
    ],
    scratch_types=[
        pltpu.VMEM((_TILE,), jnp.float32),
        pltpu.VMEM((_TILE,), jnp.float32),
        pltpu.VMEM((_TILE,), jnp.float32),
    ],
)
def _sc_rank(st_hbm, mask_hbm, ranks_hbm, s_v, mask_v, ranks_v):
    _sc_rank_body(st_hbm, mask_hbm, ranks_hbm, s_v, mask_v, ranks_v)


def kernel(routings, boosting_weights):
    scores_t = _tc_scores_t(routings, boosting_weights)
    mask_f, ranks_f = _sc_rank(scores_t.reshape(_N * _C))
    mask = jnp.transpose(mask_f.reshape(_N // _BN, _C, _BN), (0, 2, 1))
    ranks = jnp.transpose(ranks_f.reshape(_N // _BN, _C, _BN), (0, 2, 1))
    return mask.reshape(_N, _C), ranks.astype(jnp.int32).reshape(_N, _C)
